# baseline (speedup 1.0000x reference)
import jax
import jax.numpy as jnp
from jax import lax
from jax.experimental import pallas as pl
from jax.experimental.pallas import tpu as pltpu
from jax.experimental.pallas import tpu_sc as plsc

N = 10000
E = 320000
D = 128
K = 3

NC = 2
NS = 16
NPAD = 10240
DH = D // NC
EPT = E // NS
C = 80
NCHUNK = EPT // C
NB = 5
NT = NCHUNK // NB
RPT = NPAD // NS
RC = 32
NRC = RPT // RC
NG = C // 16

_f32 = jnp.float32
_i32 = jnp.int32


def _bcast_i(val):
    return jnp.full((16,), val, _i32)


_GDN = lax.GatherDimensionNumbers(offset_dims=(), collapsed_slice_dims=(0,),
                                  start_index_map=(0,))


def _lane_bcast(vec, lane):
    return lax.gather(vec, _bcast_i(lane)[:, None], dimension_numbers=_GDN,
                      slice_sizes=(1,),
                      mode=lax.GatherScatterMode.PROMISE_IN_BOUNDS)


def _sc_body(row_hbm, col_hbm, ef_hbm, x_hbm, hw_hbm,
             hid_hbm, y_hbm, b_hbm,
             idxr, idxc, dinv_ts, gbufA, gbufB, gbufC, gbufD, gbufE,
             nbuf, bbuf, hbuf, ybuf, zbuf,
             dvec, zvec, ones_c, efidxA, efidxB, bscr, hw_ts,
             acc_sh, deg_sh, dinv_sh, semA, semB, semH, semSA, semSB):
    gbufs = (gbufA, gbufB, gbufC, gbufD, gbufE)
    sems5 = (semA, semB, semH, semSA, semSB)
    sems = (semA, semB)
    ssems = (semSA, semSB)
    semL, semW = semA, semB
    c = lax.axis_index("c")
    s = lax.axis_index("s")
    ebase = s * EPT
    rbase = s * RPT
    coff = c * NPAD
    cols = c * DH

    z16 = jnp.zeros((16,), _f32)
    one16 = jnp.ones((16,), _f32)

    pltpu.sync_copy(hw_hbm, hw_ts)
    pltpu.sync_copy(row_hbm.at[s], idxr)
    pltpu.sync_copy(col_hbm.at[s], idxc)

    @pl.loop(0, RC)
    def _(r):
        for v in range(DH // 16):
            zbuf[r, pl.ds(v * 16, 16)] = z16

    @pl.loop(0, RPT // 16)
    def _(k):
        zvec[pl.ds(k * 16, 16)] = z16

    @pl.loop(0, C // 16)
    def _(k):
        ones_c[pl.ds(k * 16, 16)] = one16

    coff_v = jnp.full((16,), coff, _i32)

    @pl.loop(0, NCHUNK)
    def _(j):
        @pl.loop(0, NG)
        def _(g):
            sl = pl.ds(g * 16, 16)
            idxr[j, sl] = idxr[j, sl] + coff_v

    pltpu.sync_copy(zvec, deg_sh.at[pl.ds(rbase, RPT)])

    @pl.loop(0, NRC)
    def _(i):
        r0 = rbase + i * RC
        pltpu.sync_copy(zbuf, acc_sh.at[pl.ds(r0, RC)])

    plsc.subcore_barrier()

    with jax.named_scope("ph_hist"):
        @pl.loop(0, NCHUNK)
        def _(j):
            pltpu.async_copy(ones_c, deg_sh.at[idxc.at[j]], semH, add=True)

        @pl.loop(0, NCHUNK)
        def _(j):
            pltpu.make_async_copy(ones_c, deg_sh.at[idxc.at[0]], semH).wait()

        plsc.subcore_barrier()

    pltpu.sync_copy(deg_sh.at[pl.ds(rbase, RPT)], dvec)
    c15 = jnp.full((16,), 1.5, _f32)
    c05 = jnp.full((16,), 0.5, _f32)
    magic = jnp.full((16,), 0x5F3759DF, _i32)
    one_i = jnp.full((16,), 1, _i32)

    @pl.loop(0, RPT // 16)
    def _(k):
        sl = pl.ds(k * 16, 16)
        d = dvec[sl]
        iz = magic - lax.shift_right_logical(plsc.bitcast(d, _i32), one_i)
        z = plsc.bitcast(iz, _f32)
        for _ in range(3):
            z = z * (c15 - c05 * d * z * z)
        dvec[sl] = jnp.where(d > c05, z, z16)

    pltpu.sync_copy(dvec, dinv_sh.at[pl.ds(rbase, RPT)])
    plsc.subcore_barrier()
    pltpu.sync_copy(dinv_sh, dinv_ts)

    with jax.named_scope("ph_init"):
        hw0 = _lane_bcast(hw_ts[...], 0)

        @pl.loop(0, NRC)
        def _(i):
            r0 = rbase + i * RC
            pltpu.sync_copy(x_hbm.at[pl.ds(r0, RC), pl.ds(cols, DH)], nbuf)

            @pl.loop(0, RC // 16)
            def _(g):
                bscr[pl.ds(16, 16)] = dinv_ts[pl.ds(r0 + g * 16, 16)]

                @pl.loop(0, 16)
                def _(jr):
                    dv = plsc.load_gather(bscr, [_bcast_i(16) + jr])
                    r = g * 16 + jr
                    for v in range(DH // 16):
                        sl = pl.ds(v * 16, 16)
                        xv = nbuf[r, sl]
                        hbuf[r, sl] = hw0 * xv
                        ybuf[r, sl] = dv * xv

            pltpu.sync_copy(hbuf, hid_hbm.at[pl.ds(r0, RC), pl.ds(cols, DH)])
            pltpu.sync_copy(ybuf, y_hbm.at[pl.ds(coff + r0, RC)])

    with jax.named_scope("ph_bpass"):
        iota2 = lax.iota(_i32, 16) * 2
        efidxs = (efidxA, efidxB)

        def _ef_load(j, k):
            eix = efidxs[k]
            for g in range(NG):
                base = 2 * (ebase + j * C + g * 16) + c
                eix[pl.ds(g * 16, 16)] = iota2 + jnp.full((16,), base, _i32)
            pltpu.async_copy(ef_hbm.at[eix], gbufs[k], sems[k])

        def _ef_wait(k):
            pltpu.make_async_copy(ef_hbm.at[efidxs[0]], gbufs[k],
                                  sems[k]).wait()

        def _sct_start(j, k):
            pltpu.async_copy(gbufs[k], acc_sh.at[idxc.at[j]], ssems[k],
                             add=True)

        def _sct_wait(k):
            pltpu.make_async_copy(gbufs[k], acc_sh.at[idxc.at[0]],
                                  ssems[k]).wait()

        def _mult(j, k):
            gb = gbufs[k]

            for g in range(NG):
                sl = pl.ds(g * 16, 16)
                r16 = idxr[j, sl] - coff_v
                nr = plsc.load_gather(dinv_ts, [r16])
                for e in range(16):
                    sv = _lane_bcast(nr, e)
                    er = g * 16 + e
                    for v in range(DH // 16):
                        s2 = pl.ds(v * 16, 16)
                        gb[er, s2] = gb[er, s2] * sv

        _ef_load(0, 0)

        @pl.loop(0, NCHUNK // 2)
        def _(t):
            j0 = 2 * t
            _ef_wait(0)

            @pl.when(t > 0)
            def _():
                _sct_wait(1)

            _ef_load(j0 + 1, 1)
            _mult(j0, 0)
            _sct_start(j0, 0)
            _ef_wait(1)
            _mult(j0 + 1, 1)

            @pl.when(t + 1 < NCHUNK // 2)
            def _():
                _sct_wait(0)
                _ef_load(j0 + 2, 0)

            _sct_start(j0 + 1, 1)

        _sct_wait(0)
        _sct_wait(1)

        plsc.subcore_barrier()

    with jax.named_scope("ph_bmat"):
        @pl.loop(0, NRC)
        def _(i):
            r0 = rbase + i * RC
            pltpu.sync_copy(acc_sh.at[pl.ds(r0, RC)], bbuf)
            pltpu.sync_copy(bbuf, b_hbm.at[pl.ds(coff + r0, RC)])
            pltpu.sync_copy(zbuf, acc_sh.at[pl.ds(r0, RC)])

        plsc.subcore_barrier()

    def _y_start(j, k):
        pltpu.async_copy(y_hbm.at[idxr.at[j]], gbufs[k], sems5[k])

    def _y_wait(k):
        pltpu.make_async_copy(y_hbm.at[idxr.at[0]], gbufs[k], sems5[k]).wait()

    for h in range(1, K + 1):
        with jax.named_scope(f"ph_edge{h}"):
            for k in range(NB - 1):
                _y_start(k, k)

            @pl.loop(0, NT)
            def _(t):
                j0 = NB * t
                for k in range(NB):
                    _y_wait(k)

                    @pl.when(j0 + k + NB - 1 < NCHUNK)
                    def _(jn=j0 + k + NB - 1, kn=(k + NB - 1) % NB):
                        _y_start(jn, kn)

                    pltpu.sync_copy(gbufs[k], acc_sh.at[idxc.at[j0 + k]],
                                    add=True)

            plsc.subcore_barrier()

        with jax.named_scope(f"ph_node{h}"):
            hwv = _lane_bcast(hw_ts[...], h)

            @pl.loop(0, NRC)
            def _(i):
                r0 = rbase + i * RC
                d1 = pltpu.async_copy(acc_sh.at[pl.ds(r0, RC)], nbuf, semL)
                d2 = pltpu.async_copy(b_hbm.at[pl.ds(coff + r0, RC)], bbuf,
                                      semW)
                d3 = pltpu.async_copy(
                    hid_hbm.at[pl.ds(r0, RC), pl.ds(cols, DH)], hbuf, semH)
                d1.wait()
                d2.wait()
                d3.wait()

                @pl.loop(0, RC // 16)
                def _(g):
                    bscr[pl.ds(16, 16)] = dinv_ts[pl.ds(r0 + g * 16, 16)]

                    @pl.loop(0, 16)
                    def _(jr):
                        dv = plsc.load_gather(bscr, [_bcast_i(16) + jr])
                        r = g * 16 + jr
                        for v in range(DH // 16):
                            sl = pl.ds(v * 16, 16)
                            xv = dv * (nbuf[r, sl] + bbuf[r, sl])
                            hbuf[r, sl] = hbuf[r, sl] + hwv * xv
                            if h < K:
                                ybuf[r, sl] = dv * xv

                pltpu.sync_copy(hbuf,
                                hid_hbm.at[pl.ds(r0, RC), pl.ds(cols, DH)])
                if h < K:
                    pltpu.sync_copy(ybuf, y_hbm.at[pl.ds(coff + r0, RC)])
                    pltpu.sync_copy(zbuf, acc_sh.at[pl.ds(r0, RC)])

            plsc.subcore_barrier()


def kernel(x, edge_index, edge_feature, hopwise):
    row = edge_index[0].reshape(NS, NCHUNK, C)
    col = edge_index[1].reshape(NS, NCHUNK, C)
    xp = jnp.zeros((NPAD, D), _f32).at[:N].set(x)
    hw = jnp.zeros((16,), _f32).at[:K + 1].set(hopwise)

    mesh = plsc.VectorSubcoreMesh(core_axis_name="c", subcore_axis_name="s",
                                  num_cores=NC, num_subcores=NS)
    out_type = [jax.ShapeDtypeStruct((NPAD, D), _f32),
                jax.ShapeDtypeStruct((NC * NPAD, DH), _f32),
                jax.ShapeDtypeStruct((NC * NPAD, DH), _f32)]
    scratch = [
        pltpu.VMEM((NCHUNK, C), _i32),
        pltpu.VMEM((NCHUNK, C), _i32),
        pltpu.VMEM((NPAD,), _f32),
        pltpu.VMEM((C, DH), _f32),
        pltpu.VMEM((C, DH), _f32),
        pltpu.VMEM((C, DH), _f32),
        pltpu.VMEM((C, DH), _f32),
        pltpu.VMEM((C, DH), _f32),
        pltpu.VMEM((RC, DH), _f32),
        pltpu.VMEM((RC, DH), _f32),
        pltpu.VMEM((RC, DH), _f32),
        pltpu.VMEM((RC, DH), _f32),
        pltpu.VMEM((RC, DH), _f32),
        pltpu.VMEM((RPT,), _f32),
        pltpu.VMEM((RPT,), _f32),
        pltpu.VMEM((C,), _f32),
        pltpu.VMEM((C,), _i32),
        pltpu.VMEM((C,), _i32),
        pltpu.VMEM((32,), _f32),
        pltpu.VMEM((16,), _f32),
        pltpu.VMEM_SHARED((NPAD, DH), _f32),
        pltpu.VMEM_SHARED((NPAD,), _f32),
        pltpu.VMEM_SHARED((NPAD,), _f32),
        pltpu.SemaphoreType.DMA,
        pltpu.SemaphoreType.DMA,
        pltpu.SemaphoreType.DMA,
        pltpu.SemaphoreType.DMA,
        pltpu.SemaphoreType.DMA,
    ]
    f = pl.kernel(_sc_body, out_type=out_type, mesh=mesh,
                  scratch_types=scratch,
                  compiler_params=pltpu.CompilerParams(
                      use_tc_tiling_on_sc=False,
                      needs_layout_passes=False))
    hid, _, _ = f(row, col, edge_feature.reshape(2 * E, DH), xp, hw)
    return hid[:N]


# speedup vs baseline: 1.0248x; 1.0248x over previous
import jax
import jax.numpy as jnp
from jax import lax
from jax.experimental import pallas as pl
from jax.experimental.pallas import tpu as pltpu
from jax.experimental.pallas import tpu_sc as plsc

N = 10000
E = 320000
D = 128
K = 3

NC = 2
NS = 16
NPAD = 10240
DH = D // NC
EPT = E // NS
C = 80
NCHUNK = EPT // C
NB = 5
NT = NCHUNK // NB
RPT = NPAD // NS
RC = 32
NRC = RPT // RC
NG = C // 16

_f32 = jnp.float32
_i32 = jnp.int32


def _bcast_i(val):
    return jnp.full((16,), val, _i32)


_GDN = lax.GatherDimensionNumbers(offset_dims=(), collapsed_slice_dims=(0,),
                                  start_index_map=(0,))


def _lane_bcast(vec, lane):
    return lax.gather(vec, _bcast_i(lane)[:, None], dimension_numbers=_GDN,
                      slice_sizes=(1,),
                      mode=lax.GatherScatterMode.PROMISE_IN_BOUNDS)


def _sc_body(row_hbm, col_hbm, ef_hbm, x_hbm, hw_hbm,
             hid_hbm, y_hbm, b_hbm,
             idxr, idxc, dinv_ts, gbufA, gbufB, gbufC, gbufD, gbufE,
             nbuf, bbuf, hbuf, ybuf, zbuf,
             dvec, zvec, ones_c, efidxA, efidxB, bscr, hw_ts,
             acc_sh, deg_sh, dinv_sh, semA, semB, semH, semSA, semSB):
    gbufs = (gbufA, gbufB, gbufC, gbufD, gbufE)
    sems5 = (semA, semB, semH, semSA, semSB)
    sems = (semA, semB)
    ssems = (semSA, semSB)
    c = lax.axis_index("c")
    s = lax.axis_index("s")
    ebase = s * EPT
    rbase = s * RPT
    coff = c * NPAD
    cols = c * DH

    z16 = jnp.zeros((16,), _f32)
    one16 = jnp.ones((16,), _f32)

    pltpu.sync_copy(hw_hbm, hw_ts)
    pltpu.sync_copy(row_hbm.at[s], idxr)
    pltpu.sync_copy(col_hbm.at[s], idxc)

    @pl.loop(0, RC)
    def _(r):
        for v in range(DH // 16):
            zbuf[r, pl.ds(v * 16, 16)] = z16

    @pl.loop(0, RPT // 16)
    def _(k):
        zvec[pl.ds(k * 16, 16)] = z16

    @pl.loop(0, C // 16)
    def _(k):
        ones_c[pl.ds(k * 16, 16)] = one16

    coff_v = jnp.full((16,), coff, _i32)

    @pl.loop(0, NCHUNK)
    def _(j):
        @pl.loop(0, NG)
        def _(g):
            sl = pl.ds(g * 16, 16)
            idxr[j, sl] = idxr[j, sl] + coff_v

    pltpu.sync_copy(zvec, deg_sh.at[pl.ds(rbase, RPT)])

    @pl.loop(0, NRC)
    def _(i):
        r0 = rbase + i * RC
        pltpu.sync_copy(zbuf, acc_sh.at[pl.ds(r0, RC)])

    plsc.subcore_barrier()

    with jax.named_scope("ph_hist"):
        @pl.loop(0, NCHUNK)
        def _(j):
            pltpu.async_copy(ones_c, deg_sh.at[idxc.at[j]], semH, add=True)

        @pl.loop(0, NCHUNK)
        def _(j):
            pltpu.make_async_copy(ones_c, deg_sh.at[idxc.at[0]], semH).wait()

        plsc.subcore_barrier()

    pltpu.sync_copy(deg_sh.at[pl.ds(rbase, RPT)], dvec)
    c15 = jnp.full((16,), 1.5, _f32)
    c05 = jnp.full((16,), 0.5, _f32)
    magic = jnp.full((16,), 0x5F3759DF, _i32)
    one_i = jnp.full((16,), 1, _i32)

    @pl.loop(0, RPT // 16)
    def _(k):
        sl = pl.ds(k * 16, 16)
        d = dvec[sl]
        iz = magic - lax.shift_right_logical(plsc.bitcast(d, _i32), one_i)
        z = plsc.bitcast(iz, _f32)
        for _ in range(3):
            z = z * (c15 - c05 * d * z * z)
        dvec[sl] = jnp.where(d > c05, z, z16)

    pltpu.sync_copy(dvec, dinv_sh.at[pl.ds(rbase, RPT)])
    plsc.subcore_barrier()
    pltpu.sync_copy(dinv_sh, dinv_ts)

    with jax.named_scope("ph_init"):
        bscr[pl.ds(16, 16)] = hw_ts[...]
        hw0 = plsc.load_gather(bscr, [_bcast_i(16)])

        @pl.loop(0, NRC)
        def _(i):
            r0 = rbase + i * RC
            pltpu.sync_copy(x_hbm.at[pl.ds(r0, RC), pl.ds(cols, DH)], nbuf)

            @pl.loop(0, RC // 16)
            def _(g):
                bscr[pl.ds(16, 16)] = dinv_ts[pl.ds(r0 + g * 16, 16)]
                for jr in range(16):
                    dv = plsc.load_gather(bscr, [_bcast_i(16 + jr)])
                    r = g * 16 + jr
                    for v in range(DH // 16):
                        sl = pl.ds(v * 16, 16)
                        xv = nbuf[r, sl]
                        hbuf[r, sl] = hw0 * xv
                        ybuf[r, sl] = dv * xv

            pltpu.sync_copy(hbuf, hid_hbm.at[pl.ds(r0, RC), pl.ds(cols, DH)])
            pltpu.sync_copy(ybuf, y_hbm.at[pl.ds(coff + r0, RC)])

    with jax.named_scope("ph_bpass"):
        iota2 = lax.iota(_i32, 16) * 2
        efidxs = (efidxA, efidxB)

        def _ef_load(j, k):
            eix = efidxs[k]
            for g in range(NG):
                base = 2 * (ebase + j * C + g * 16) + c
                eix[pl.ds(g * 16, 16)] = iota2 + jnp.full((16,), base, _i32)
            pltpu.async_copy(ef_hbm.at[eix], gbufs[k], sems[k])

        def _ef_wait(k):
            pltpu.make_async_copy(ef_hbm.at[efidxs[0]], gbufs[k],
                                  sems[k]).wait()

        def _sct_start(j, k):
            pltpu.async_copy(gbufs[k], acc_sh.at[idxc.at[j]], ssems[k],
                             add=True)

        def _sct_wait(k):
            pltpu.make_async_copy(gbufs[k], acc_sh.at[idxc.at[0]],
                                  ssems[k]).wait()

        def _mult(j, k):
            gb = gbufs[k]

            for g in range(NG):
                sl = pl.ds(g * 16, 16)
                r16 = idxr[j, sl] - coff_v
                nr = plsc.load_gather(dinv_ts, [r16])
                for e in range(16):
                    sv = _lane_bcast(nr, e)
                    er = g * 16 + e
                    for v in range(DH // 16):
                        s2 = pl.ds(v * 16, 16)
                        gb[er, s2] = gb[er, s2] * sv

        _ef_load(0, 0)

        @pl.loop(0, NCHUNK // 2)
        def _(t):
            j0 = 2 * t
            _ef_wait(0)

            @pl.when(t > 0)
            def _():
                _sct_wait(1)

            _ef_load(j0 + 1, 1)
            _mult(j0, 0)
            _sct_start(j0, 0)
            _ef_wait(1)
            _mult(j0 + 1, 1)

            @pl.when(t + 1 < NCHUNK // 2)
            def _():
                _sct_wait(0)
                _ef_load(j0 + 2, 0)

            _sct_start(j0 + 1, 1)

        _sct_wait(0)
        _sct_wait(1)

        plsc.subcore_barrier()

    with jax.named_scope("ph_bmat"):
        @pl.loop(0, NRC)
        def _(i):
            r0 = rbase + i * RC
            pltpu.sync_copy(acc_sh.at[pl.ds(r0, RC)], bbuf)
            pltpu.sync_copy(bbuf, b_hbm.at[pl.ds(coff + r0, RC)])
            pltpu.sync_copy(zbuf, acc_sh.at[pl.ds(r0, RC)])

        plsc.subcore_barrier()

    def _y_start(j, k):
        pltpu.async_copy(y_hbm.at[idxr.at[j]], gbufs[k], sems5[k])

    def _y_wait(k):
        pltpu.make_async_copy(y_hbm.at[idxr.at[0]], gbufs[k], sems5[k]).wait()

    for h in range(1, K + 1):
        with jax.named_scope(f"ph_edge{h}"):
            for k in range(NB - 1):
                _y_start(k, k)

            @pl.loop(0, NT)
            def _(t):
                j0 = NB * t
                for k in range(NB):
                    _y_wait(k)

                    @pl.when(j0 + k + NB - 1 < NCHUNK)
                    def _(jn=j0 + k + NB - 1, kn=(k + NB - 1) % NB):
                        _y_start(jn, kn)

                    pltpu.sync_copy(gbufs[k], acc_sh.at[idxc.at[j0 + k]],
                                    add=True)

            plsc.subcore_barrier()

        with jax.named_scope(f"ph_node{h}"):
            bscr[pl.ds(16, 16)] = hw_ts[...]
            hwv = plsc.load_gather(bscr, [_bcast_i(16 + h)])

            @pl.loop(0, NRC)
            def _(i):
                r0 = rbase + i * RC
                pltpu.sync_copy(acc_sh.at[pl.ds(r0, RC)], nbuf)
                pltpu.sync_copy(b_hbm.at[pl.ds(coff + r0, RC)], bbuf)
                pltpu.sync_copy(hid_hbm.at[pl.ds(r0, RC), pl.ds(cols, DH)],
                                hbuf)

                @pl.loop(0, RC // 16)
                def _(g):
                    bscr[pl.ds(16, 16)] = dinv_ts[pl.ds(r0 + g * 16, 16)]
                    for jr in range(16):
                        dv = plsc.load_gather(bscr, [_bcast_i(16 + jr)])
                        r = g * 16 + jr
                        for v in range(DH // 16):
                            sl = pl.ds(v * 16, 16)
                            xv = dv * (nbuf[r, sl] + bbuf[r, sl])
                            hbuf[r, sl] = hbuf[r, sl] + hwv * xv
                            if h < K:
                                ybuf[r, sl] = dv * xv

                pltpu.sync_copy(hbuf,
                                hid_hbm.at[pl.ds(r0, RC), pl.ds(cols, DH)])
                if h < K:
                    pltpu.sync_copy(ybuf, y_hbm.at[pl.ds(coff + r0, RC)])
                    pltpu.sync_copy(zbuf, acc_sh.at[pl.ds(r0, RC)])

            plsc.subcore_barrier()


def kernel(x, edge_index, edge_feature, hopwise):
    row = edge_index[0].reshape(NS, NCHUNK, C)
    col = edge_index[1].reshape(NS, NCHUNK, C)
    xp = jnp.zeros((NPAD, D), _f32).at[:N].set(x)
    hw = jnp.zeros((16,), _f32).at[:K + 1].set(hopwise)

    mesh = plsc.VectorSubcoreMesh(core_axis_name="c", subcore_axis_name="s",
                                  num_cores=NC, num_subcores=NS)
    out_type = [jax.ShapeDtypeStruct((NPAD, D), _f32),
                jax.ShapeDtypeStruct((NC * NPAD, DH), _f32),
                jax.ShapeDtypeStruct((NC * NPAD, DH), _f32)]
    scratch = [
        pltpu.VMEM((NCHUNK, C), _i32),
        pltpu.VMEM((NCHUNK, C), _i32),
        pltpu.VMEM((NPAD,), _f32),
        pltpu.VMEM((C, DH), _f32),
        pltpu.VMEM((C, DH), _f32),
        pltpu.VMEM((C, DH), _f32),
        pltpu.VMEM((C, DH), _f32),
        pltpu.VMEM((C, DH), _f32),
        pltpu.VMEM((RC, DH), _f32),
        pltpu.VMEM((RC, DH), _f32),
        pltpu.VMEM((RC, DH), _f32),
        pltpu.VMEM((RC, DH), _f32),
        pltpu.VMEM((RC, DH), _f32),
        pltpu.VMEM((RPT,), _f32),
        pltpu.VMEM((RPT,), _f32),
        pltpu.VMEM((C,), _f32),
        pltpu.VMEM((C,), _i32),
        pltpu.VMEM((C,), _i32),
        pltpu.VMEM((32,), _f32),
        pltpu.VMEM((16,), _f32),
        pltpu.VMEM_SHARED((NPAD, DH), _f32),
        pltpu.VMEM_SHARED((NPAD,), _f32),
        pltpu.VMEM_SHARED((NPAD,), _f32),
        pltpu.SemaphoreType.DMA,
        pltpu.SemaphoreType.DMA,
        pltpu.SemaphoreType.DMA,
        pltpu.SemaphoreType.DMA,
        pltpu.SemaphoreType.DMA,
    ]
    f = pl.kernel(_sc_body, out_type=out_type, mesh=mesh,
                  scratch_types=scratch,
                  compiler_params=pltpu.CompilerParams(
                      use_tc_tiling_on_sc=False,
                      needs_layout_passes=False))
    hid, _, _ = f(row, col, edge_feature.reshape(2 * E, DH), xp, hw)
    return hid[:N]


# speedup vs baseline: 1.1144x; 1.0874x over previous
import jax
import jax.numpy as jnp
from jax import lax
from jax.experimental import pallas as pl
from jax.experimental.pallas import tpu as pltpu
from jax.experimental.pallas import tpu_sc as plsc

N = 10000
E = 320000
D = 128
K = 3

NC = 2
NS = 16
NPAD = 10240
DH = D // NC
EPT = E // NS
C = 80
NCHUNK = EPT // C
NB = 5
NT = NCHUNK // NB
RPT = NPAD // NS
RC = 32
NRC = RPT // RC
NG = C // 16

_f32 = jnp.float32
_i32 = jnp.int32


def _bcast_i(val):
    return jnp.full((16,), val, _i32)


_GDN = lax.GatherDimensionNumbers(offset_dims=(), collapsed_slice_dims=(0,),
                                  start_index_map=(0,))


def _lane_bcast(vec, lane):
    return lax.gather(vec, _bcast_i(lane)[:, None], dimension_numbers=_GDN,
                      slice_sizes=(1,),
                      mode=lax.GatherScatterMode.PROMISE_IN_BOUNDS)


def _sc_body(row_hbm, col_hbm, ef_hbm, x_hbm, hw_hbm,
             hid_hbm, y_hbm, b_hbm,
             idxr, idxc, dinv_ts, gbufA, gbufB, gbufC, gbufD, gbufE,
             nbuf, bbuf, hbuf, ybuf, zbuf,
             dvec, zvec, ones_c, efidxA, efidxB, efidxC, efidxD, efidxE,
             bscr, hw_ts,
             acc_sh, deg_sh, dinv_sh, semA, semB, semH, semSA, semSB,
             semX0, semX1, semX2, semX3, semX4):
    gbufs = (gbufA, gbufB, gbufC, gbufD, gbufE)
    sems5 = (semA, semB, semH, semSA, semSB)
    xsems = (semX0, semX1, semX2, semX3, semX4)
    c = lax.axis_index("c")
    s = lax.axis_index("s")
    ebase = s * EPT
    rbase = s * RPT
    coff = c * NPAD
    cols = c * DH

    z16 = jnp.zeros((16,), _f32)
    one16 = jnp.ones((16,), _f32)

    pltpu.sync_copy(hw_hbm, hw_ts)
    pltpu.sync_copy(row_hbm.at[s], idxr)
    pltpu.sync_copy(col_hbm.at[s], idxc)

    @pl.loop(0, RC)
    def _(r):
        for v in range(DH // 16):
            zbuf[r, pl.ds(v * 16, 16)] = z16

    @pl.loop(0, RPT // 16)
    def _(k):
        zvec[pl.ds(k * 16, 16)] = z16

    @pl.loop(0, C // 16)
    def _(k):
        ones_c[pl.ds(k * 16, 16)] = one16

    coff_v = jnp.full((16,), coff, _i32)

    @pl.loop(0, NCHUNK)
    def _(j):
        @pl.loop(0, NG)
        def _(g):
            sl = pl.ds(g * 16, 16)
            idxr[j, sl] = idxr[j, sl] + coff_v

    pltpu.sync_copy(zvec, deg_sh.at[pl.ds(rbase, RPT)])

    @pl.loop(0, NRC)
    def _(i):
        r0 = rbase + i * RC
        pltpu.sync_copy(zbuf, acc_sh.at[pl.ds(r0, RC)])

    plsc.subcore_barrier()

    with jax.named_scope("ph_hist"):
        @pl.loop(0, NCHUNK)
        def _(j):
            pltpu.async_copy(ones_c, deg_sh.at[idxc.at[j]], semH, add=True)

        @pl.loop(0, NCHUNK)
        def _(j):
            pltpu.make_async_copy(ones_c, deg_sh.at[idxc.at[0]], semH).wait()

        plsc.subcore_barrier()

    pltpu.sync_copy(deg_sh.at[pl.ds(rbase, RPT)], dvec)
    c15 = jnp.full((16,), 1.5, _f32)
    c05 = jnp.full((16,), 0.5, _f32)
    magic = jnp.full((16,), 0x5F3759DF, _i32)
    one_i = jnp.full((16,), 1, _i32)

    @pl.loop(0, RPT // 16)
    def _(k):
        sl = pl.ds(k * 16, 16)
        d = dvec[sl]
        iz = magic - lax.shift_right_logical(plsc.bitcast(d, _i32), one_i)
        z = plsc.bitcast(iz, _f32)
        for _ in range(3):
            z = z * (c15 - c05 * d * z * z)
        dvec[sl] = jnp.where(d > c05, z, z16)

    pltpu.sync_copy(dvec, dinv_sh.at[pl.ds(rbase, RPT)])
    plsc.subcore_barrier()
    pltpu.sync_copy(dinv_sh, dinv_ts)

    with jax.named_scope("ph_init"):
        bscr[pl.ds(16, 16)] = hw_ts[...]
        hw0 = plsc.load_gather(bscr, [_bcast_i(16)])

        @pl.loop(0, NRC)
        def _(i):
            r0 = rbase + i * RC
            pltpu.sync_copy(x_hbm.at[pl.ds(r0, RC), pl.ds(cols, DH)], nbuf)

            @pl.loop(0, RC // 16)
            def _(g):
                bscr[pl.ds(16, 16)] = dinv_ts[pl.ds(r0 + g * 16, 16)]
                for jr in range(16):
                    dv = plsc.load_gather(bscr, [_bcast_i(16 + jr)])
                    r = g * 16 + jr
                    for v in range(DH // 16):
                        sl = pl.ds(v * 16, 16)
                        xv = nbuf[r, sl]
                        hbuf[r, sl] = hw0 * xv
                        ybuf[r, sl] = dv * xv

            pltpu.sync_copy(hbuf, hid_hbm.at[pl.ds(r0, RC), pl.ds(cols, DH)])
            pltpu.sync_copy(ybuf, y_hbm.at[pl.ds(coff + r0, RC)])

    with jax.named_scope("ph_bpass"):
        iota2 = lax.iota(_i32, 16) * 2
        efidxs = (efidxA, efidxB, efidxC, efidxD, efidxE)

        def _ef_load(j, k):
            eix = efidxs[k]
            for g in range(NG):
                base = 2 * (ebase + j * C + g * 16) + c
                eix[pl.ds(g * 16, 16)] = iota2 + jnp.full((16,), base, _i32)
            pltpu.async_copy(ef_hbm.at[eix], gbufs[k], sems5[k])

        def _ef_wait(k):
            pltpu.make_async_copy(ef_hbm.at[efidxs[k]], gbufs[k],
                                  sems5[k]).wait()

        def _sct_start(j, k):
            pltpu.async_copy(gbufs[k], acc_sh.at[idxc.at[j]], xsems[k],
                             add=True)

        def _sct_wait(k):
            pltpu.make_async_copy(gbufs[k], acc_sh.at[idxc.at[0]],
                                  xsems[k]).wait()

        def _mult(j, k):
            gb = gbufs[k]

            for g in range(NG):
                sl = pl.ds(g * 16, 16)
                r16 = idxr[j, sl] - coff_v
                nr = plsc.load_gather(dinv_ts, [r16])
                for e in range(16):
                    sv = _lane_bcast(nr, e)
                    er = g * 16 + e
                    for v in range(DH // 16):
                        s2 = pl.ds(v * 16, 16)
                        gb[er, s2] = gb[er, s2] * sv

        for k in range(NB - 1):
            _ef_load(k, k)

        @pl.loop(0, NT)
        def _(t):
            j0 = NB * t
            for k in range(NB):
                _ef_wait(k)
                jn = j0 + k + NB - 1
                kn = (k + NB - 1) % NB

                @pl.when(jn < NCHUNK)
                def _(jn=jn, kn=kn):
                    @pl.when(jn >= NB)
                    def _():
                        _sct_wait(kn)

                    _ef_load(jn, kn)

                _mult(j0 + k, k)
                _sct_start(j0 + k, k)

        for k in range(NB):
            _sct_wait(k)

        plsc.subcore_barrier()

    with jax.named_scope("ph_bmat"):
        @pl.loop(0, NRC)
        def _(i):
            r0 = rbase + i * RC
            pltpu.sync_copy(acc_sh.at[pl.ds(r0, RC)], bbuf)
            pltpu.sync_copy(bbuf, b_hbm.at[pl.ds(coff + r0, RC)])
            pltpu.sync_copy(zbuf, acc_sh.at[pl.ds(r0, RC)])

        plsc.subcore_barrier()

    def _y_start(j, k):
        pltpu.async_copy(y_hbm.at[idxr.at[j]], gbufs[k], sems5[k])

    def _y_wait(k):
        pltpu.make_async_copy(y_hbm.at[idxr.at[0]], gbufs[k], sems5[k]).wait()

    for h in range(1, K + 1):
        with jax.named_scope(f"ph_edge{h}"):
            for k in range(NB - 1):
                _y_start(k, k)

            @pl.loop(0, NT)
            def _(t):
                j0 = NB * t
                for k in range(NB):
                    _y_wait(k)

                    @pl.when(j0 + k + NB - 1 < NCHUNK)
                    def _(jn=j0 + k + NB - 1, kn=(k + NB - 1) % NB):
                        _y_start(jn, kn)

                    pltpu.sync_copy(gbufs[k], acc_sh.at[idxc.at[j0 + k]],
                                    add=True)

            plsc.subcore_barrier()

        with jax.named_scope(f"ph_node{h}"):
            bscr[pl.ds(16, 16)] = hw_ts[...]
            hwv = plsc.load_gather(bscr, [_bcast_i(16 + h)])

            @pl.loop(0, NRC)
            def _(i):
                r0 = rbase + i * RC
                pltpu.sync_copy(acc_sh.at[pl.ds(r0, RC)], nbuf)
                pltpu.sync_copy(b_hbm.at[pl.ds(coff + r0, RC)], bbuf)
                pltpu.sync_copy(hid_hbm.at[pl.ds(r0, RC), pl.ds(cols, DH)],
                                hbuf)

                @pl.loop(0, RC // 16)
                def _(g):
                    bscr[pl.ds(16, 16)] = dinv_ts[pl.ds(r0 + g * 16, 16)]
                    for jr in range(16):
                        dv = plsc.load_gather(bscr, [_bcast_i(16 + jr)])
                        r = g * 16 + jr
                        for v in range(DH // 16):
                            sl = pl.ds(v * 16, 16)
                            xv = dv * (nbuf[r, sl] + bbuf[r, sl])
                            hbuf[r, sl] = hbuf[r, sl] + hwv * xv
                            if h < K:
                                ybuf[r, sl] = dv * xv

                pltpu.sync_copy(hbuf,
                                hid_hbm.at[pl.ds(r0, RC), pl.ds(cols, DH)])
                if h < K:
                    pltpu.sync_copy(ybuf, y_hbm.at[pl.ds(coff + r0, RC)])
                    pltpu.sync_copy(zbuf, acc_sh.at[pl.ds(r0, RC)])

            plsc.subcore_barrier()


def kernel(x, edge_index, edge_feature, hopwise):
    row = edge_index[0].reshape(NS, NCHUNK, C)
    col = edge_index[1].reshape(NS, NCHUNK, C)
    xp = jnp.zeros((NPAD, D), _f32).at[:N].set(x)
    hw = jnp.zeros((16,), _f32).at[:K + 1].set(hopwise)

    mesh = plsc.VectorSubcoreMesh(core_axis_name="c", subcore_axis_name="s",
                                  num_cores=NC, num_subcores=NS)
    out_type = [jax.ShapeDtypeStruct((NPAD, D), _f32),
                jax.ShapeDtypeStruct((NC * NPAD, DH), _f32),
                jax.ShapeDtypeStruct((NC * NPAD, DH), _f32)]
    scratch = [
        pltpu.VMEM((NCHUNK, C), _i32),
        pltpu.VMEM((NCHUNK, C), _i32),
        pltpu.VMEM((NPAD,), _f32),
        pltpu.VMEM((C, DH), _f32),
        pltpu.VMEM((C, DH), _f32),
        pltpu.VMEM((C, DH), _f32),
        pltpu.VMEM((C, DH), _f32),
        pltpu.VMEM((C, DH), _f32),
        pltpu.VMEM((RC, DH), _f32),
        pltpu.VMEM((RC, DH), _f32),
        pltpu.VMEM((RC, DH), _f32),
        pltpu.VMEM((RC, DH), _f32),
        pltpu.VMEM((RC, DH), _f32),
        pltpu.VMEM((RPT,), _f32),
        pltpu.VMEM((RPT,), _f32),
        pltpu.VMEM((C,), _f32),
        pltpu.VMEM((C,), _i32),
        pltpu.VMEM((C,), _i32),
        pltpu.VMEM((C,), _i32),
        pltpu.VMEM((C,), _i32),
        pltpu.VMEM((C,), _i32),
        pltpu.VMEM((32,), _f32),
        pltpu.VMEM((16,), _f32),
        pltpu.VMEM_SHARED((NPAD, DH), _f32),
        pltpu.VMEM_SHARED((NPAD,), _f32),
        pltpu.VMEM_SHARED((NPAD,), _f32),
        pltpu.SemaphoreType.DMA,
        pltpu.SemaphoreType.DMA,
        pltpu.SemaphoreType.DMA,
        pltpu.SemaphoreType.DMA,
        pltpu.SemaphoreType.DMA,
        pltpu.SemaphoreType.DMA,
        pltpu.SemaphoreType.DMA,
        pltpu.SemaphoreType.DMA,
        pltpu.SemaphoreType.DMA,
        pltpu.SemaphoreType.DMA,
    ]
    f = pl.kernel(_sc_body, out_type=out_type, mesh=mesh,
                  scratch_types=scratch,
                  compiler_params=pltpu.CompilerParams(
                      use_tc_tiling_on_sc=False,
                      needs_layout_passes=False))
    hid, _, _ = f(row, col, edge_feature.reshape(2 * E, DH), xp, hw)
    return hid[:N]


# speedup vs baseline: 1.1986x; 1.0756x over previous
import jax
import jax.numpy as jnp
from jax import lax
from jax.experimental import pallas as pl
from jax.experimental.pallas import tpu as pltpu
from jax.experimental.pallas import tpu_sc as plsc

N = 10000
E = 320000
D = 128
K = 3

NC = 2
NS = 16
NPAD = 10240
DH = D // NC
EPT = E // NS
C = 80
NCHUNK = EPT // C
NB = 5
NT = NCHUNK // NB
RPT = NPAD // NS
RC = 32
NRC = RPT // RC
NG = C // 16

_f32 = jnp.float32
_i32 = jnp.int32


def _bcast_i(val):
    return jnp.full((16,), val, _i32)


_GDN = lax.GatherDimensionNumbers(offset_dims=(), collapsed_slice_dims=(0,),
                                  start_index_map=(0,))


def _lane_bcast(vec, lane):
    return lax.gather(vec, _bcast_i(lane)[:, None], dimension_numbers=_GDN,
                      slice_sizes=(1,),
                      mode=lax.GatherScatterMode.PROMISE_IN_BOUNDS)


def _sc_body(row_hbm, col_hbm, ef_hbm, x_hbm, hw_hbm,
             hid_hbm, y_hbm, b_hbm,
             idxr, idxc, dinv_ts, gbufA, gbufB, gbufC, gbufD, gbufE,
             nbuf, bbuf, hbuf, ybuf, zbuf,
             dvec, zvec, ones_c, efidxA, efidxB, efidxC, efidxD, efidxE,
             bscr, hw_ts,
             acc_sh, deg_sh, dinv_sh, semA, semB, semH, semSA, semSB,
             semX0, semX1, semX2, semX3, semX4):
    gbufs = (gbufA, gbufB, gbufC, gbufD, gbufE)
    sems5 = (semA, semB, semH, semSA, semSB)
    xsems = (semX0, semX1, semX2, semX3, semX4)
    c = lax.axis_index("c")
    s = lax.axis_index("s")
    ebase = s * EPT
    rbase = s * RPT
    coff = c * NPAD
    cols = c * DH

    z16 = jnp.zeros((16,), _f32)
    one16 = jnp.ones((16,), _f32)

    pltpu.sync_copy(hw_hbm, hw_ts)
    pltpu.sync_copy(row_hbm.at[s], idxr)
    pltpu.sync_copy(col_hbm.at[s], idxc)

    @pl.loop(0, RC)
    def _(r):
        for v in range(DH // 16):
            zbuf[r, pl.ds(v * 16, 16)] = z16

    @pl.loop(0, RPT // 16)
    def _(k):
        zvec[pl.ds(k * 16, 16)] = z16

    @pl.loop(0, C // 16)
    def _(k):
        ones_c[pl.ds(k * 16, 16)] = one16

    coff_v = jnp.full((16,), coff, _i32)

    @pl.loop(0, NCHUNK)
    def _(j):
        @pl.loop(0, NG)
        def _(g):
            sl = pl.ds(g * 16, 16)
            idxr[j, sl] = idxr[j, sl] + coff_v

    pltpu.sync_copy(zvec, deg_sh.at[pl.ds(rbase, RPT)])

    @pl.loop(0, NRC)
    def _(i):
        r0 = rbase + i * RC
        pltpu.sync_copy(zbuf, acc_sh.at[pl.ds(r0, RC)])

    plsc.subcore_barrier()

    with jax.named_scope("ph_hist"):
        @pl.loop(0, NCHUNK)
        def _(j):
            pltpu.async_copy(ones_c, deg_sh.at[idxc.at[j]], semH, add=True)

        @pl.loop(0, NCHUNK)
        def _(j):
            pltpu.make_async_copy(ones_c, deg_sh.at[idxc.at[0]], semH).wait()

        plsc.subcore_barrier()

    pltpu.sync_copy(deg_sh.at[pl.ds(rbase, RPT)], dvec)
    c15 = jnp.full((16,), 1.5, _f32)
    c05 = jnp.full((16,), 0.5, _f32)
    magic = jnp.full((16,), 0x5F3759DF, _i32)
    one_i = jnp.full((16,), 1, _i32)

    @pl.loop(0, RPT // 16)
    def _(k):
        sl = pl.ds(k * 16, 16)
        d = dvec[sl]
        iz = magic - lax.shift_right_logical(plsc.bitcast(d, _i32), one_i)
        z = plsc.bitcast(iz, _f32)
        for _ in range(3):
            z = z * (c15 - c05 * d * z * z)
        dvec[sl] = jnp.where(d > c05, z, z16)

    pltpu.sync_copy(dvec, dinv_sh.at[pl.ds(rbase, RPT)])
    plsc.subcore_barrier()
    pltpu.sync_copy(dinv_sh, dinv_ts)

    with jax.named_scope("ph_init"):
        bscr[pl.ds(16, 16)] = hw_ts[...]
        hw0 = plsc.load_gather(bscr, [_bcast_i(16)])

        @pl.loop(0, NRC)
        def _(i):
            r0 = rbase + i * RC
            pltpu.sync_copy(x_hbm.at[pl.ds(r0, RC), pl.ds(cols, DH)], nbuf)

            @pl.loop(0, RC // 16)
            def _(g):
                bscr[pl.ds(16, 16)] = dinv_ts[pl.ds(r0 + g * 16, 16)]
                for jr in range(16):
                    dv = plsc.load_gather(bscr, [_bcast_i(16 + jr)])
                    r = g * 16 + jr
                    for v in range(DH // 16):
                        sl = pl.ds(v * 16, 16)
                        xv = nbuf[r, sl]
                        hbuf[r, sl] = hw0 * xv
                        ybuf[r, sl] = dv * xv

            pltpu.sync_copy(hbuf, hid_hbm.at[pl.ds(r0, RC), pl.ds(cols, DH)])
            pltpu.sync_copy(ybuf, y_hbm.at[pl.ds(coff + r0, RC)])

    with jax.named_scope("ph_bpass"):
        iota2 = lax.iota(_i32, 16) * 2
        efidxs = (efidxA, efidxB, efidxC, efidxD, efidxE)

        def _ef_load(j, k):
            eix = efidxs[k]
            for g in range(NG):
                base = 2 * (ebase + j * C + g * 16) + c
                eix[pl.ds(g * 16, 16)] = iota2 + jnp.full((16,), base, _i32)
            pltpu.async_copy(ef_hbm.at[eix], gbufs[k], sems5[k])

        def _ef_wait(k):
            pltpu.make_async_copy(ef_hbm.at[efidxs[k]], gbufs[k],
                                  sems5[k]).wait()

        def _sct_start(j, k):
            pltpu.async_copy(gbufs[k], acc_sh.at[idxc.at[j]], xsems[k],
                             add=True)

        def _sct_wait(k):
            pltpu.make_async_copy(gbufs[k], acc_sh.at[idxc.at[0]],
                                  xsems[k]).wait()

        def _mult(j, k):
            gb = gbufs[k]

            for g in range(NG):
                sl = pl.ds(g * 16, 16)
                r16 = idxr[j, sl] - coff_v
                nr = plsc.load_gather(dinv_ts, [r16])
                for e in range(16):
                    sv = _lane_bcast(nr, e)
                    er = g * 16 + e
                    for v in range(DH // 16):
                        s2 = pl.ds(v * 16, 16)
                        gb[er, s2] = gb[er, s2] * sv

        for k in range(NB - 1):
            _ef_load(k, k)

        @pl.loop(0, NT)
        def _(t):
            j0 = NB * t
            for k in range(NB):
                _ef_wait(k)
                jn = j0 + k + NB - 1
                kn = (k + NB - 1) % NB

                @pl.when(jn < NCHUNK)
                def _(jn=jn, kn=kn):
                    @pl.when(jn >= NB)
                    def _():
                        _sct_wait(kn)

                    _ef_load(jn, kn)

                _mult(j0 + k, k)
                _sct_start(j0 + k, k)

        for k in range(NB):
            _sct_wait(k)

        plsc.subcore_barrier()

    with jax.named_scope("ph_bmat"):
        @pl.loop(0, NRC)
        def _(i):
            r0 = rbase + i * RC
            pltpu.sync_copy(acc_sh.at[pl.ds(r0, RC)], bbuf)
            pltpu.sync_copy(bbuf, b_hbm.at[pl.ds(coff + r0, RC)])
            pltpu.sync_copy(zbuf, acc_sh.at[pl.ds(r0, RC)])

        plsc.subcore_barrier()

    def _y_start(j, k):
        pltpu.async_copy(y_hbm.at[idxr.at[j]], gbufs[k], sems5[k])

    def _y_wait(k):
        pltpu.make_async_copy(y_hbm.at[idxr.at[0]], gbufs[k], sems5[k]).wait()

    for h in range(1, K + 1):
        with jax.named_scope(f"ph_edge{h}"):
            for k in range(NB - 1):
                _y_start(k, k)

            @pl.loop(0, NT)
            def _(t):
                j0 = NB * t
                for k in range(NB):
                    _y_wait(k)

                    @pl.when(j0 + k + NB - 1 < NCHUNK)
                    def _(jn=j0 + k + NB - 1, kn=(k + NB - 1) % NB):
                        _y_start(jn, kn)

                    pltpu.sync_copy(gbufs[k], acc_sh.at[idxc.at[j0 + k]],
                                    add=True)

            plsc.subcore_barrier()

        with jax.named_scope(f"ph_node{h}"):
            bscr[pl.ds(16, 16)] = hw_ts[...]
            hwv = plsc.load_gather(bscr, [_bcast_i(16 + h)])

            @pl.loop(0, NRC)
            def _(i):
                r0 = rbase + i * RC
                d1 = pltpu.async_copy(acc_sh.at[pl.ds(r0, RC)], nbuf, semX0)
                d2 = pltpu.async_copy(b_hbm.at[pl.ds(coff + r0, RC)], bbuf,
                                      semX1)
                d3 = pltpu.async_copy(
                    hid_hbm.at[pl.ds(r0, RC), pl.ds(cols, DH)], hbuf, semX2)
                d1.wait()
                d2.wait()
                d3.wait()

                @pl.loop(0, RC // 16)
                def _(g):
                    bscr[pl.ds(16, 16)] = dinv_ts[pl.ds(r0 + g * 16, 16)]
                    for jr in range(16):
                        dv = plsc.load_gather(bscr, [_bcast_i(16 + jr)])
                        r = g * 16 + jr
                        for v in range(DH // 16):
                            sl = pl.ds(v * 16, 16)
                            xv = dv * (nbuf[r, sl] + bbuf[r, sl])
                            hbuf[r, sl] = hbuf[r, sl] + hwv * xv
                            if h < K:
                                ybuf[r, sl] = dv * xv

                pltpu.sync_copy(hbuf,
                                hid_hbm.at[pl.ds(r0, RC), pl.ds(cols, DH)])
                if h < K:
                    pltpu.sync_copy(ybuf, y_hbm.at[pl.ds(coff + r0, RC)])
                    pltpu.sync_copy(zbuf, acc_sh.at[pl.ds(r0, RC)])

            plsc.subcore_barrier()


def kernel(x, edge_index, edge_feature, hopwise):
    row = edge_index[0].reshape(NS, NCHUNK, C)
    col = edge_index[1].reshape(NS, NCHUNK, C)
    xp = jnp.zeros((NPAD, D), _f32).at[:N].set(x)
    hw = jnp.zeros((16,), _f32).at[:K + 1].set(hopwise)

    mesh = plsc.VectorSubcoreMesh(core_axis_name="c", subcore_axis_name="s",
                                  num_cores=NC, num_subcores=NS)
    out_type = [jax.ShapeDtypeStruct((NPAD, D), _f32),
                jax.ShapeDtypeStruct((NC * NPAD, DH), _f32),
                jax.ShapeDtypeStruct((NC * NPAD, DH), _f32)]
    scratch = [
        pltpu.VMEM((NCHUNK, C), _i32),
        pltpu.VMEM((NCHUNK, C), _i32),
        pltpu.VMEM((NPAD,), _f32),
        pltpu.VMEM((C, DH), _f32),
        pltpu.VMEM((C, DH), _f32),
        pltpu.VMEM((C, DH), _f32),
        pltpu.VMEM((C, DH), _f32),
        pltpu.VMEM((C, DH), _f32),
        pltpu.VMEM((RC, DH), _f32),
        pltpu.VMEM((RC, DH), _f32),
        pltpu.VMEM((RC, DH), _f32),
        pltpu.VMEM((RC, DH), _f32),
        pltpu.VMEM((RC, DH), _f32),
        pltpu.VMEM((RPT,), _f32),
        pltpu.VMEM((RPT,), _f32),
        pltpu.VMEM((C,), _f32),
        pltpu.VMEM((C,), _i32),
        pltpu.VMEM((C,), _i32),
        pltpu.VMEM((C,), _i32),
        pltpu.VMEM((C,), _i32),
        pltpu.VMEM((C,), _i32),
        pltpu.VMEM((32,), _f32),
        pltpu.VMEM((16,), _f32),
        pltpu.VMEM_SHARED((NPAD, DH), _f32),
        pltpu.VMEM_SHARED((NPAD,), _f32),
        pltpu.VMEM_SHARED((NPAD,), _f32),
        pltpu.SemaphoreType.DMA,
        pltpu.SemaphoreType.DMA,
        pltpu.SemaphoreType.DMA,
        pltpu.SemaphoreType.DMA,
        pltpu.SemaphoreType.DMA,
        pltpu.SemaphoreType.DMA,
        pltpu.SemaphoreType.DMA,
        pltpu.SemaphoreType.DMA,
        pltpu.SemaphoreType.DMA,
        pltpu.SemaphoreType.DMA,
    ]
    f = pl.kernel(_sc_body, out_type=out_type, mesh=mesh,
                  scratch_types=scratch,
                  compiler_params=pltpu.CompilerParams(
                      use_tc_tiling_on_sc=False,
                      needs_layout_passes=False))
    hid, _, _ = f(row, col, edge_feature.reshape(2 * E, DH), xp, hw)
    return hid[:N]
